# Initial kernel scaffold; baseline (speedup 1.0000x reference)
import jax
import jax.numpy as jnp
from jax.experimental import pallas as pl


def kernel(mask, weights):
    raise NotImplementedError("write your pallas kernel here")


# speedup vs baseline: 2.4885x; 2.4885x over previous
import functools

import jax
import jax.numpy as jnp
from jax import lax
from jax.experimental import pallas as pl
from jax.experimental.pallas import tpu as pltpu
from jax.experimental.pallas import tpu_sc as plsc

_VOCAB = 1000000
_EMB = 16
_BATCH = 16384
_HIST = 200

_B = _BATCH * _HIST
_NW = 32
_IDXW = 128
_ROWS_PER_BLOCK = 16
_BLK = _ROWS_PER_BLOCK * _IDXW
_N_IDX_ROWS = _B // _IDXW
_ROWS_PER_W = _N_IDX_ROWS // _NW
_BLOCKS_PER_W = _ROWS_PER_W // _ROWS_PER_BLOCK


def _emb_kernel(idx_hbm, table_hbm, out_hbm, idx_v, rows_v, sem):
    nc = 2
    wid = lax.axis_index("s") * nc + lax.axis_index("c")
    row_base = wid * _ROWS_PER_W

    def block(b, carry):
        row0 = row_base + b * _ROWS_PER_BLOCK
        pltpu.sync_copy(idx_hbm.at[pl.ds(row0, _ROWS_PER_BLOCK)], idx_v)
        copies = []
        for j in range(_ROWS_PER_BLOCK):
            copies.append(
                pltpu.async_copy(
                    table_hbm.at[idx_v.at[j]],
                    rows_v.at[pl.ds(j * _IDXW, _IDXW)],
                    sem,
                )
            )
        for c in copies:
            c.wait()
        pltpu.sync_copy(rows_v, out_hbm.at[pl.ds(row0 * _IDXW, _BLK)])
        return carry

    lax.fori_loop(0, _BLOCKS_PER_W, block, 0)


@jax.jit
def kernel(mask, weights):
    idx = mask.reshape(_N_IDX_ROWS, _IDXW).astype(jnp.int32)
    mesh = plsc.VectorSubcoreMesh(core_axis_name="c", subcore_axis_name="s")
    k = functools.partial(
        pl.kernel,
        mesh=mesh,
        out_type=jax.ShapeDtypeStruct((_B, _EMB), jnp.float32),
        scratch_types=[
            pltpu.VMEM((_ROWS_PER_BLOCK, _IDXW), jnp.int32),
            pltpu.VMEM((_BLK, _EMB), jnp.float32),
            pltpu.SemaphoreType.DMA,
        ],
        compiler_params=pltpu.CompilerParams(use_tc_tiling_on_sc=False),
    )(_emb_kernel)
    out = k(idx, weights)
    return out.reshape(_BATCH, _HIST, _EMB)


# speedup vs baseline: 2.5335x; 1.0181x over previous
import functools

import jax
import jax.numpy as jnp
from jax import lax
from jax.experimental import pallas as pl
from jax.experimental.pallas import tpu as pltpu
from jax.experimental.pallas import tpu_sc as plsc

_VOCAB = 1000000
_EMB = 16
_BATCH = 16384
_HIST = 200

_B = _BATCH * _HIST
_NW = 32
_IDXW = 128
_ROWS_PER_BLOCK = 16
_BLK = _ROWS_PER_BLOCK * _IDXW
_N_IDX_ROWS = _B // _IDXW
_ROWS_PER_W = _N_IDX_ROWS // _NW
_BLOCKS_PER_W = _ROWS_PER_W // _ROWS_PER_BLOCK


def _emb_kernel(idx_hbm, table_hbm, out_hbm, idx_v, rows_v, gsem, ssems):
    nc = 2
    wid = lax.axis_index("s") * nc + lax.axis_index("c")
    row_base = wid * _ROWS_PER_W

    def process(b, slot, wait_store):
        row0 = row_base + b * _ROWS_PER_BLOCK
        if wait_store:
            pltpu.make_async_copy(
                rows_v.at[slot], out_hbm.at[pl.ds(0, _BLK)], ssems.at[slot]
            ).wait()
        pltpu.sync_copy(idx_hbm.at[pl.ds(row0, _ROWS_PER_BLOCK)], idx_v.at[slot])
        copies = []
        for j in range(_ROWS_PER_BLOCK):
            copies.append(
                pltpu.async_copy(
                    table_hbm.at[idx_v.at[slot].at[j]],
                    rows_v.at[slot].at[pl.ds(j * _IDXW, _IDXW)],
                    gsem,
                )
            )
        for c in copies:
            c.wait()
        pltpu.async_copy(
            rows_v.at[slot], out_hbm.at[pl.ds(row0 * _IDXW, _BLK)], ssems.at[slot]
        )

    for s in range(2):
        process(s, s, False)

    def outer(g, carry):
        for s in range(2):
            process(2 * g + s, s, True)
        return carry

    lax.fori_loop(1, _BLOCKS_PER_W // 2, outer, 0)

    for s in range(2):
        pltpu.make_async_copy(
            rows_v.at[s], out_hbm.at[pl.ds(0, _BLK)], ssems.at[s]
        ).wait()


@jax.jit
def kernel(mask, weights):
    idx = mask.reshape(_N_IDX_ROWS, _IDXW).astype(jnp.int32)
    mesh = plsc.VectorSubcoreMesh(core_axis_name="c", subcore_axis_name="s")
    k = functools.partial(
        pl.kernel,
        mesh=mesh,
        out_type=jax.ShapeDtypeStruct((_B, _EMB), jnp.float32),
        scratch_types=[
            pltpu.VMEM((2, _ROWS_PER_BLOCK, _IDXW), jnp.int32),
            pltpu.VMEM((2, _BLK, _EMB), jnp.float32),
            pltpu.SemaphoreType.DMA,
            pltpu.SemaphoreType.DMA((2,)),
        ],
        compiler_params=pltpu.CompilerParams(use_tc_tiling_on_sc=False),
    )(_emb_kernel)
    out = k(idx, weights)
    return out.reshape(_BATCH, _HIST, _EMB)
